# Initial kernel scaffold; baseline (speedup 1.0000x reference)
import jax
import jax.numpy as jnp
from jax.experimental import pallas as pl


def kernel(x, embed_table, keys_hv, therm):
    raise NotImplementedError("write your pallas kernel here")


# speedup vs baseline: 17.7427x; 17.7427x over previous
import functools

import jax
import jax.numpy as jnp
from jax import lax
from jax.experimental import pallas as pl
from jax.experimental.pallas import tpu as pltpu
from jax.experimental.pallas import tpu_sc as plsc

LOW = -2.0
HIGH = 2.0
D = 64

NC, NS = 2, 16
NW = NC * NS
CHUNK = 128


def _sc_gather(x3, table):
    R = x3.shape[0] * x3.shape[1]
    rpw = x3.shape[1]
    mesh = plsc.VectorSubcoreMesh(core_axis_name="c", subcore_axis_name="s")

    @functools.partial(
        pl.kernel,
        out_type=jax.ShapeDtypeStruct((R, CHUNK, D), jnp.float32),
        mesh=mesh,
        scratch_types=[
            pltpu.VMEM((rpw, CHUNK), jnp.int32),
            pltpu.VMEM((rpw, CHUNK, D), jnp.float32),
            pltpu.SemaphoreType.DMA,
        ],
        compiler_params=pltpu.CompilerParams(use_tc_tiling_on_sc=False),
    )
    def k(table_hbm, x_hbm, out_hbm, idx_v, rows_v, sem):
        w = lax.axis_index("s") * NC + lax.axis_index("c")
        base = w * rpw
        pltpu.sync_copy(x_hbm.at[w], idx_v)
        copies = [
            pltpu.async_copy(table_hbm.at[idx_v.at[c]], rows_v.at[c], sem)
            for c in range(rpw)
        ]
        for cp in copies:
            cp.wait()
        pltpu.sync_copy(rows_v, out_hbm.at[pl.ds(base, rpw)])

    return k(table, x3)


def _tc_compute(base2, keys):
    N = base2.shape[0]
    T = 256
    scale = (D + 1 - 1) / (HIGH - LOW)

    def body(b_ref, k_ref, o_ref):
        b = b_ref[...]
        idx = jnp.clip(jnp.round((b - LOW) * scale), 0.0, float(D)).astype(jnp.int32)
        j = lax.broadcasted_iota(jnp.int32, (T, D, D), 2)
        m = j < idx[:, :, None]
        kk = k_ref[...][None, :, :]
        o_ref[...] = jnp.sum(jnp.where(m, kk, -kk), axis=1)

    return pl.pallas_call(
        body,
        grid=(N // T,),
        in_specs=[
            pl.BlockSpec((T, D), lambda i: (i, 0)),
            pl.BlockSpec((D, D), lambda i: (0, 0)),
        ],
        out_specs=pl.BlockSpec((T, D), lambda i: (i, 0)),
        out_shape=jax.ShapeDtypeStruct((N, D), jnp.float32),
    )(base2, keys)


def kernel(x, embed_table, keys_hv, therm):
    B, L = x.shape
    N = B * L
    x3 = x.reshape(NW, N // (NW * CHUNK), CHUNK).astype(jnp.int32)
    base3 = _sc_gather(x3, embed_table)
    out2 = _tc_compute(base3.reshape(N, D), keys_hv)
    return out2.reshape(B, L, D)


# speedup vs baseline: 37.1942x; 2.0963x over previous
import functools

import jax
import jax.numpy as jnp
from jax import lax
from jax.experimental import pallas as pl
from jax.experimental.pallas import tpu as pltpu
from jax.experimental.pallas import tpu_sc as plsc

LOW = -2.0
HIGH = 2.0
D = 64

NC, NS = 2, 16
NW = NC * NS
CHUNK = 128


def _sc_gather(x3, table):
    R = x3.shape[0] * x3.shape[1]
    rpw = x3.shape[1]
    mesh = plsc.VectorSubcoreMesh(core_axis_name="c", subcore_axis_name="s")

    @functools.partial(
        pl.kernel,
        out_type=jax.ShapeDtypeStruct((R, CHUNK, D), jnp.float32),
        mesh=mesh,
        scratch_types=[
            pltpu.VMEM((rpw, CHUNK), jnp.int32),
            pltpu.VMEM((rpw, CHUNK, D), jnp.float32),
            pltpu.SemaphoreType.DMA,
        ],
        compiler_params=pltpu.CompilerParams(use_tc_tiling_on_sc=False),
    )
    def k(table_hbm, x_hbm, out_hbm, idx_v, rows_v, sem):
        w = lax.axis_index("s") * NC + lax.axis_index("c")
        base = w * rpw
        pltpu.sync_copy(x_hbm.at[w], idx_v)
        copies = [
            pltpu.async_copy(table_hbm.at[idx_v.at[c]], rows_v.at[c], sem)
            for c in range(rpw)
        ]
        for cp in copies:
            cp.wait()
        pltpu.sync_copy(rows_v, out_hbm.at[pl.ds(base, rpw)])

    return k(table, x3)


def _tc_compute(base2, keys, n_l, n_b):
    scale = (D + 1 - 1) / (HIGH - LOW)

    def body(b_ref, k_ref, o_ref):
        y = (b_ref[...] - LOW) * scale
        yt = y.T
        kv = k_ref[...]
        rows = []
        for j in range(D):
            m = yt > (j + 0.5)
            kj = kv[:, j:j + 1]
            rows.append(jnp.sum(jnp.where(m, kj, -kj), axis=0, keepdims=True))
        o_ref[0] = jnp.concatenate(rows, axis=0)

    return pl.pallas_call(
        body,
        grid=(n_l,),
        in_specs=[
            pl.BlockSpec((n_b, D), lambda i: (i, 0)),
            pl.BlockSpec((D, D), lambda i: (0, 0)),
        ],
        out_specs=pl.BlockSpec((1, D, n_b), lambda i: (i, 0, 0)),
        out_shape=jax.ShapeDtypeStruct((n_l, D, n_b), jnp.float32),
    )(base2, keys)


def kernel(x, embed_table, keys_hv, therm):
    B, L = x.shape
    N = B * L
    x3 = x.T.reshape(NW, N // (NW * CHUNK), CHUNK).astype(jnp.int32)
    base3 = _sc_gather(x3, embed_table)
    out3 = _tc_compute(base3.reshape(N, D), keys_hv, L, B)
    return jnp.transpose(out3, (2, 0, 1))


# speedup vs baseline: 41.2038x; 1.1078x over previous
import functools

import jax
import jax.numpy as jnp
from jax import lax
from jax.experimental import pallas as pl
from jax.experimental.pallas import tpu as pltpu
from jax.experimental.pallas import tpu_sc as plsc

LOW = -2.0
HIGH = 2.0
D = 64

NC, NS = 2, 16
NW = NC * NS
CHUNK = 128


def _sc_gather(x1, tablep):
    N = x1.shape[0]
    R = N // CHUNK
    rpw = R // NW
    npw = N // NW
    mesh = plsc.VectorSubcoreMesh(core_axis_name="c", subcore_axis_name="s")

    @functools.partial(
        pl.kernel,
        out_type=jax.ShapeDtypeStruct((R, CHUNK, 2 * D), jnp.float32),
        mesh=mesh,
        scratch_types=[
            pltpu.VMEM((npw,), jnp.int32),
            pltpu.VMEM((rpw, CHUNK, 2 * D), jnp.float32),
            pltpu.SemaphoreType.DMA,
        ],
        compiler_params=pltpu.CompilerParams(use_tc_tiling_on_sc=True),
    )
    def k(table_hbm, x_hbm, out_hbm, idx_v, rows_v, sem):
        w = lax.axis_index("s") * NC + lax.axis_index("c")
        pltpu.sync_copy(x_hbm.at[pl.ds(w * npw, npw)], idx_v)
        copies = [
            pltpu.async_copy(
                table_hbm.at[idx_v.at[pl.ds(c * CHUNK, CHUNK)]],
                rows_v.at[c],
                sem,
            )
            for c in range(rpw)
        ]
        for cp in copies:
            cp.wait()
        pltpu.sync_copy(rows_v, out_hbm.at[pl.ds(w * rpw, rpw)])

    return k(tablep, x1)


def _tc_compute(base2, keys, n_l, n_b):
    scale = (D + 1 - 1) / (HIGH - LOW)

    def body(b_ref, k_ref, o_ref):
        y = (b_ref[:, :D] - LOW) * scale
        yt = y.T
        kv = k_ref[...]
        rows = []
        for j in range(D):
            m = yt > (j + 0.5)
            kj = kv[:, j:j + 1]
            rows.append(jnp.sum(jnp.where(m, kj, -kj), axis=0, keepdims=True))
        o_ref[0] = jnp.concatenate(rows, axis=0)

    return pl.pallas_call(
        body,
        grid=(n_l,),
        in_specs=[
            pl.BlockSpec((n_b, 2 * D), lambda i: (i, 0)),
            pl.BlockSpec((D, D), lambda i: (0, 0)),
        ],
        out_specs=pl.BlockSpec((1, D, n_b), lambda i: (i, 0, 0)),
        out_shape=jax.ShapeDtypeStruct((n_l, D, n_b), jnp.float32),
    )(base2, keys)


def kernel(x, embed_table, keys_hv, therm):
    B, L = x.shape
    N = B * L
    x1 = x.T.reshape(N).astype(jnp.int32)
    tablep = jnp.pad(embed_table, ((0, 0), (0, D)))
    base3 = _sc_gather(x1, tablep)
    out3 = _tc_compute(base3.reshape(N, 2 * D), keys_hv, L, B)
    return jnp.transpose(out3, (2, 0, 1))


# speedup vs baseline: 44.4072x; 1.0777x over previous
import functools

import jax
import jax.numpy as jnp
from jax import lax
from jax.experimental import pallas as pl
from jax.experimental.pallas import tpu as pltpu
from jax.experimental.pallas import tpu_sc as plsc

LOW = -2.0
HIGH = 2.0
D = 64

NC, NS = 2, 16
NW = NC * NS
CHUNK = 128


def _sc_gather(x1, tablep):
    N = x1.shape[0]
    R = N // CHUNK
    rpw = R // NW
    npw = N // NW
    mesh = plsc.VectorSubcoreMesh(core_axis_name="c", subcore_axis_name="s")

    @functools.partial(
        pl.kernel,
        out_type=jax.ShapeDtypeStruct((R, CHUNK, 2 * D), jnp.float32),
        mesh=mesh,
        scratch_types=[
            pltpu.VMEM((npw,), jnp.int32),
            pltpu.VMEM((rpw, CHUNK, 2 * D), jnp.float32),
            pltpu.SemaphoreType.DMA,
        ],
        compiler_params=pltpu.CompilerParams(use_tc_tiling_on_sc=True),
    )
    def k(table_hbm, x_hbm, out_hbm, idx_v, rows_v, sem):
        w = lax.axis_index("s") * NC + lax.axis_index("c")
        pltpu.sync_copy(x_hbm.at[pl.ds(w * npw, npw)], idx_v)
        copies = [
            pltpu.async_copy(
                table_hbm.at[idx_v.at[pl.ds(c * CHUNK, CHUNK)]],
                rows_v.at[c],
                sem,
            )
            for c in range(rpw)
        ]
        for cp in copies:
            cp.wait()
        pltpu.sync_copy(rows_v, out_hbm.at[pl.ds(w * rpw, rpw)])

    return k(tablep, x1)


def _tc_compute(base2, keys, n_l, n_b):
    scale = (D + 1 - 1) / (HIGH - LOW)

    def body(b_ref, k_ref, o_ref):
        y = (b_ref[:, :D] - LOW) * scale
        yt = y.T
        kv = k_ref[...]
        rows = []
        for j in range(D):
            sig = jnp.where(yt > (j + 0.5), 1.0, -1.0)
            rows.append(
                lax.dot_general(kv[:, j:j + 1], sig,
                                (((0,), (0,)), ((), ())),
                                preferred_element_type=jnp.float32))
        o_ref[0] = jnp.concatenate(rows, axis=0)

    return pl.pallas_call(
        body,
        grid=(n_l,),
        in_specs=[
            pl.BlockSpec((n_b, 2 * D), lambda i: (i, 0)),
            pl.BlockSpec((D, D), lambda i: (0, 0)),
        ],
        out_specs=pl.BlockSpec((1, D, n_b), lambda i: (i, 0, 0)),
        out_shape=jax.ShapeDtypeStruct((n_l, D, n_b), jnp.float32),
    )(base2, keys)


def kernel(x, embed_table, keys_hv, therm):
    B, L = x.shape
    N = B * L
    x1 = x.T.reshape(N).astype(jnp.int32)
    tablep = jnp.pad(embed_table, ((0, 0), (0, D)))
    base3 = _sc_gather(x1, tablep)
    out3 = _tc_compute(base3.reshape(N, 2 * D), keys_hv, L, B)
    return jnp.transpose(out3, (2, 0, 1))


# speedup vs baseline: 45.5658x; 1.0261x over previous
import functools

import jax
import jax.numpy as jnp
from jax import lax
from jax.experimental import pallas as pl
from jax.experimental.pallas import tpu as pltpu
from jax.experimental.pallas import tpu_sc as plsc

LOW = -2.0
HIGH = 2.0
D = 64

NC, NS = 2, 16
NW = NC * NS
CHUNK = 128


def _sc_gather(x1, tablep):
    N = x1.shape[0]
    R = N // CHUNK
    rpw = R // NW
    npw = N // NW
    mesh = plsc.VectorSubcoreMesh(core_axis_name="c", subcore_axis_name="s")

    @functools.partial(
        pl.kernel,
        out_type=jax.ShapeDtypeStruct((R, CHUNK, 2 * D), jnp.float32),
        mesh=mesh,
        scratch_types=[
            pltpu.VMEM((npw,), jnp.int32),
            pltpu.VMEM((rpw, CHUNK, 2 * D), jnp.float32),
            pltpu.SemaphoreType.DMA,
        ],
        compiler_params=pltpu.CompilerParams(use_tc_tiling_on_sc=True),
    )
    def k(table_hbm, x_hbm, out_hbm, idx_v, rows_v, sem):
        w = lax.axis_index("s") * NC + lax.axis_index("c")
        pltpu.sync_copy(x_hbm.at[pl.ds(w * npw, npw)], idx_v)
        copies = [
            pltpu.async_copy(
                table_hbm.at[idx_v.at[pl.ds(c * CHUNK, CHUNK)]],
                rows_v.at[c],
                sem,
            )
            for c in range(rpw)
        ]
        for cp in copies:
            cp.wait()
        pltpu.sync_copy(rows_v, out_hbm.at[pl.ds(w * rpw, rpw)])

    return k(tablep, x1)


def _tc_compute(base2, keys, n_l, n_b):
    scale = (D + 1 - 1) / (HIGH - LOW)

    GL = 2

    def body(b_ref, k_ref, o_ref):
        kv = k_ref[...]
        for g in range(GL):
            b = b_ref[pl.ds(g * n_b, n_b), :D]
            yt = ((b - LOW) * scale).T
            rows = []
            for j in range(D):
                sig = jnp.where(yt > (j + 0.5), 1.0, -1.0)
                rows.append(
                    lax.dot_general(kv[:, j:j + 1], sig,
                                    (((0,), (0,)), ((), ())),
                                    preferred_element_type=jnp.float32))
            o_ref[g] = jnp.concatenate(rows, axis=0)

    return pl.pallas_call(
        body,
        grid=(n_l // GL,),
        in_specs=[
            pl.BlockSpec((GL * n_b, 2 * D), lambda i: (i, 0)),
            pl.BlockSpec((D, D), lambda i: (0, 0)),
        ],
        out_specs=pl.BlockSpec((GL, D, n_b), lambda i: (i, 0, 0)),
        out_shape=jax.ShapeDtypeStruct((n_l, D, n_b), jnp.float32),
    )(base2, keys)


def kernel(x, embed_table, keys_hv, therm):
    B, L = x.shape
    N = B * L
    x1 = x.T.reshape(N).astype(jnp.int32)
    tablep = jnp.pad(embed_table, ((0, 0), (0, D)))
    base3 = _sc_gather(x1, tablep)
    out3 = _tc_compute(base3.reshape(N, 2 * D), keys_hv, L, B)
    return jnp.transpose(out3, (2, 0, 1))


# speedup vs baseline: 45.9517x; 1.0085x over previous
import functools

import jax
import jax.numpy as jnp
from jax import lax
from jax.experimental import pallas as pl
from jax.experimental.pallas import tpu as pltpu
from jax.experimental.pallas import tpu_sc as plsc

LOW = -2.0
HIGH = 2.0
D = 64

NC, NS = 2, 16
NW = NC * NS
CHUNK = 128


def _sc_gather(x1, tablep):
    N = x1.shape[0]
    R = N // CHUNK
    rpw = R // NW
    npw = N // NW
    mesh = plsc.VectorSubcoreMesh(core_axis_name="c", subcore_axis_name="s")

    @functools.partial(
        pl.kernel,
        out_type=jax.ShapeDtypeStruct((R, CHUNK, 2 * D), jnp.float32),
        mesh=mesh,
        scratch_types=[
            pltpu.VMEM((npw,), jnp.int32),
            pltpu.VMEM((rpw, CHUNK, 2 * D), jnp.float32),
            pltpu.SemaphoreType.DMA,
        ],
        compiler_params=pltpu.CompilerParams(use_tc_tiling_on_sc=True),
    )
    def k(table_hbm, x_hbm, out_hbm, idx_v, rows_v, sem):
        w = lax.axis_index("s") * NC + lax.axis_index("c")
        pltpu.sync_copy(x_hbm.at[pl.ds(w * npw, npw)], idx_v)
        copies = [
            pltpu.async_copy(
                table_hbm.at[idx_v.at[pl.ds(c * CHUNK, CHUNK)]],
                rows_v.at[c],
                sem,
            )
            for c in range(rpw)
        ]
        for cp in copies:
            cp.wait()
        pltpu.sync_copy(rows_v, out_hbm.at[pl.ds(w * rpw, rpw)])

    return k(tablep, x1)


def _tc_compute(base2, keys, n_l, n_b):
    scale = (D + 1 - 1) / (HIGH - LOW)

    GL = 4

    def body(b_ref, k_ref, o_ref):
        kv = k_ref[...]
        for g in range(GL):
            b = b_ref[pl.ds(g * n_b, n_b), :D]
            yt = ((b - LOW) * scale).T
            rows = []
            for j in range(D):
                sig = jnp.where(yt > (j + 0.5), 1.0, -1.0)
                rows.append(
                    lax.dot_general(kv[:, j:j + 1], sig,
                                    (((0,), (0,)), ((), ())),
                                    preferred_element_type=jnp.float32))
            o_ref[g] = jnp.concatenate(rows, axis=0)

    return pl.pallas_call(
        body,
        grid=(n_l // GL,),
        in_specs=[
            pl.BlockSpec((GL * n_b, 2 * D), lambda i: (i, 0)),
            pl.BlockSpec((D, D), lambda i: (0, 0)),
        ],
        out_specs=pl.BlockSpec((GL, D, n_b), lambda i: (i, 0, 0)),
        out_shape=jax.ShapeDtypeStruct((n_l, D, n_b), jnp.float32),
    )(base2, keys)


def kernel(x, embed_table, keys_hv, therm):
    B, L = x.shape
    N = B * L
    x1 = x.T.reshape(N).astype(jnp.int32)
    tablep = jnp.pad(embed_table, ((0, 0), (0, D)))
    base3 = _sc_gather(x1, tablep)
    out3 = _tc_compute(base3.reshape(N, 2 * D), keys_hv, L, B)
    return jnp.transpose(out3, (2, 0, 1))


# speedup vs baseline: 45.9918x; 1.0009x over previous
import functools

import jax
import jax.numpy as jnp
from jax import lax
from jax.experimental import pallas as pl
from jax.experimental.pallas import tpu as pltpu
from jax.experimental.pallas import tpu_sc as plsc

LOW = -2.0
HIGH = 2.0
D = 64

NC, NS = 2, 16
NW = NC * NS
CHUNK = 128


def _sc_gather(x1, tablep):
    N = x1.shape[0]
    R = N // CHUNK
    rpw = R // NW
    npw = N // NW
    mesh = plsc.VectorSubcoreMesh(core_axis_name="c", subcore_axis_name="s")

    @functools.partial(
        pl.kernel,
        out_type=jax.ShapeDtypeStruct((R, CHUNK, 2 * D), jnp.float32),
        mesh=mesh,
        scratch_types=[
            pltpu.VMEM((npw,), jnp.int32),
            pltpu.VMEM((rpw, CHUNK, 2 * D), jnp.float32),
            pltpu.SemaphoreType.DMA,
        ],
        compiler_params=pltpu.CompilerParams(use_tc_tiling_on_sc=True),
    )
    def k(table_hbm, x_hbm, out_hbm, idx_v, rows_v, sem):
        w = lax.axis_index("s") * NC + lax.axis_index("c")
        pltpu.sync_copy(x_hbm.at[pl.ds(w * npw, npw)], idx_v)
        copies = [
            pltpu.async_copy(
                table_hbm.at[idx_v.at[pl.ds(c * CHUNK, CHUNK)]],
                rows_v.at[c],
                sem,
            )
            for c in range(rpw)
        ]
        for cp in copies:
            cp.wait()
        pltpu.sync_copy(rows_v, out_hbm.at[pl.ds(w * rpw, rpw)])

    return k(tablep, x1)


def _tc_compute(base2, keys, n_l, n_b):
    scale = (D + 1 - 1) / (HIGH - LOW)

    GL = 5

    def body(b_ref, k_ref, o_ref):
        kv = k_ref[...]
        for g in range(GL):
            b = b_ref[pl.ds(g * n_b, n_b), :D]
            yt = ((b - LOW) * scale).T
            rows = []
            for j in range(D):
                sig = jnp.where(yt > (j + 0.5), 1.0, -1.0)
                rows.append(
                    lax.dot_general(kv[:, j:j + 1], sig,
                                    (((0,), (0,)), ((), ())),
                                    preferred_element_type=jnp.float32))
            o_ref[g] = jnp.concatenate(rows, axis=0)

    return pl.pallas_call(
        body,
        grid=(n_l // GL,),
        in_specs=[
            pl.BlockSpec((GL * n_b, 2 * D), lambda i: (i, 0)),
            pl.BlockSpec((D, D), lambda i: (0, 0)),
        ],
        out_specs=pl.BlockSpec((GL, D, n_b), lambda i: (i, 0, 0)),
        out_shape=jax.ShapeDtypeStruct((n_l, D, n_b), jnp.float32),
    )(base2, keys)


def kernel(x, embed_table, keys_hv, therm):
    B, L = x.shape
    N = B * L
    x1 = x.T.reshape(N).astype(jnp.int32)
    tablep = jnp.pad(embed_table, ((0, 0), (0, D)))
    base3 = _sc_gather(x1, tablep)
    out3 = _tc_compute(base3.reshape(N, 2 * D), keys_hv, L, B)
    return jnp.transpose(out3, (2, 0, 1))
